# Initial kernel scaffold; baseline (speedup 1.0000x reference)
import jax
import jax.numpy as jnp
from jax.experimental import pallas as pl


def kernel(output, label, index, acc_loss_array):
    raise NotImplementedError("write your pallas kernel here")


# speedup vs baseline: 3.6900x; 3.6900x over previous
import jax
import jax.numpy as jnp
from jax.experimental import pallas as pl
from jax.experimental.pallas import tpu as pltpu

_LAM = 0.2
_LANE = 128


def _make_main(B, C, R, TH, n_examp, interpret=False):
    T = R // TH
    inv_total = 1.0 / (B * R * _LANE)

    def body(idx_ref, bsrc_ref, x_ref, lab_ref, acc_ref, loss_ref, S, G):
        t = pl.program_id(0)
        b = pl.program_id(1)

        m = x_ref[0, 0]
        for c in range(1, C):
            m = jnp.maximum(m, x_ref[0, c])
        lab = lab_ref[0]
        s = jnp.zeros_like(m)
        xl = jnp.zeros_like(m)
        for c in range(C):
            xc = x_ref[0, c]
            s = s + jnp.exp(xc - m)
            xl = jnp.where(lab == c, xc, xl)
        tl = m + jnp.log(s) - xl

        S[b] = tl
        G[b] = acc_ref[0]

        @pl.when(jnp.logical_and(t == 0, b == 0))
        def _init():
            loss_ref[0] = 0.0

        @pl.when(b == B - 1)
        def _reduce():
            part = jnp.zeros((TH, _LANE), jnp.float32)
            for bp in range(B):
                tls = S[bsrc_ref[bp]]
                den = _LAM + (1.0 - _LAM) * (G[bp] + tls)
                part = part + S[bp] / den
            loss_ref[0] += jnp.sum(part) * inv_total

    grid_spec = pltpu.PrefetchScalarGridSpec(
        num_scalar_prefetch=2,
        grid=(T, B),
        in_specs=[
            pl.BlockSpec((1, C, TH, _LANE), lambda t, b, idx, bsrc: (b, 0, t, 0)),
            pl.BlockSpec((1, TH, _LANE), lambda t, b, idx, bsrc: (b, t, 0)),
            pl.BlockSpec((1, TH, _LANE), lambda t, b, idx, bsrc: (idx[b], t, 0)),
        ],
        out_specs=pl.BlockSpec(memory_space=pltpu.SMEM),
        scratch_shapes=[
            pltpu.VMEM((B, TH, _LANE), jnp.float32),
            pltpu.VMEM((B, TH, _LANE), jnp.float32),
        ],
    )
    return pl.pallas_call(
        body,
        grid_spec=grid_spec,
        out_shape=jax.ShapeDtypeStruct((1,), jnp.float32),
        interpret=interpret,
    )


def kernel(output, label, index, acc_loss_array, interpret=False):
    B, C, H, W = output.shape
    HW = H * W
    R = HW // _LANE
    TH = 56
    out4 = output.reshape(B, C, R, _LANE)
    lab3 = label.astype(jnp.int32).reshape(B, R, _LANE)
    acc3 = acc_loss_array.reshape(acc_loss_array.shape[0], R, _LANE)
    idx = index.astype(jnp.int32)
    eq = idx[:, None] == idx[None, :]
    bsrc = jnp.max(
        jnp.where(eq, jnp.arange(B, dtype=jnp.int32)[None, :], -1), axis=1
    )
    loss = _make_main(B, C, R, TH, acc_loss_array.shape[0], interpret=interpret)(
        idx, bsrc, out4, lab3, acc3
    )
    return loss[0]


# speedup vs baseline: 3.9630x; 1.0740x over previous
import jax
import jax.numpy as jnp
from jax.experimental import pallas as pl
from jax.experimental.pallas import tpu as pltpu

_LAM = 0.2


def _make_main(B, C, H, W, TH, interpret=False):
    T = H // TH
    inv_total = 1.0 / (B * H * W)

    def body(idx_ref, bsrc_ref, x_ref, lab_ref, acc_ref, loss_ref, S, G):
        t = pl.program_id(0)
        b = pl.program_id(1)

        m = x_ref[0, 0]
        for c in range(1, C):
            m = jnp.maximum(m, x_ref[0, c])
        lab = lab_ref[0]
        s = jnp.zeros_like(m)
        xl = jnp.zeros_like(m)
        for c in range(C):
            xc = x_ref[0, c]
            s = s + jnp.exp(xc - m)
            xl = jnp.where(lab == c, xc, xl)
        tl = m + jnp.log(s) - xl

        S[b] = tl
        G[b] = acc_ref[0]

        @pl.when(jnp.logical_and(t == 0, b == 0))
        def _init():
            loss_ref[0] = 0.0

        @pl.when(b == B - 1)
        def _reduce():
            part = jnp.zeros((TH, W), jnp.float32)
            for bp in range(B):
                tls = S[bsrc_ref[bp]]
                den = _LAM + (1.0 - _LAM) * (G[bp] + tls)
                part = part + S[bp] / den
            loss_ref[0] += jnp.sum(part) * inv_total

    grid_spec = pltpu.PrefetchScalarGridSpec(
        num_scalar_prefetch=2,
        grid=(T, B),
        in_specs=[
            pl.BlockSpec((1, C, TH, W), lambda t, b, idx, bsrc: (b, 0, t, 0)),
            pl.BlockSpec((1, TH, W), lambda t, b, idx, bsrc: (b, t, 0)),
            pl.BlockSpec((1, TH, W), lambda t, b, idx, bsrc: (idx[b], t, 0)),
        ],
        out_specs=pl.BlockSpec(memory_space=pltpu.SMEM),
        scratch_shapes=[
            pltpu.VMEM((B, TH, W), jnp.float32),
            pltpu.VMEM((B, TH, W), jnp.float32),
        ],
    )
    return pl.pallas_call(
        body,
        grid_spec=grid_spec,
        out_shape=jax.ShapeDtypeStruct((1,), jnp.float32),
        interpret=interpret,
    )


def kernel(output, label, index, acc_loss_array, interpret=False):
    B, C, H, W = output.shape
    TH = 32
    lab = label.astype(jnp.int32)
    idx = index.astype(jnp.int32)
    eq = idx[:, None] == idx[None, :]
    bsrc = jnp.max(
        jnp.where(eq, jnp.arange(B, dtype=jnp.int32)[None, :], -1), axis=1
    )
    loss = _make_main(B, C, H, W, TH, interpret=interpret)(
        idx, bsrc, output, lab, acc_loss_array
    )
    return loss[0]
